# Initial kernel scaffold; baseline (speedup 1.0000x reference)
import jax
import jax.numpy as jnp
from jax.experimental import pallas as pl


def kernel(x, edge_index, edge_attr, params):
    raise NotImplementedError("write your pallas kernel here")


# speedup vs baseline: 1.2756x; 1.2756x over previous
import jax
import jax.numpy as jnp
from jax import lax
from jax.experimental import pallas as pl
from jax.experimental.pallas import tpu as pltpu
from jax.experimental.pallas import tpu_sc as plsc

N = 10000
E = 160000
DE = 16
C = 128
CA = 64
NT = 32
K = 128
NCH = 40
EPT = NCH * K
EPAD = NT * EPT
NPADE = EPAD - E
NACC = 10240
RPT = NACC // 16
BN = 400
GN = N // BN


def _sc_gather(hbs, srcr, din):
    cbs = din // C
    mesh = plsc.VectorSubcoreMesh(core_axis_name="c", subcore_axis_name="s")

    def body(*refs):
        hb = refs[:cbs]
        srcr_ref = refs[cbs]
        outs = refs[cbs + 1:2 * cbs + 1]
        sidx, gbuf, sem = refs[2 * cbs + 1:]

        cid = lax.axis_index("c")
        sid = lax.axis_index("s")
        wid = cid * 16 + sid
        row0_base = wid * EPT
        for cb in range(cbs):
            def chunk(ch, carry):
                pltpu.sync_copy(srcr_ref.at[wid, ch], sidx)
                pltpu.async_copy(hb[cb].at[sidx], gbuf, sem).wait()
                pltpu.sync_copy(
                    gbuf, outs[cb].at[pl.ds(row0_base + ch * K, K)])
                return carry
            lax.fori_loop(0, NCH, chunk, 0)

    fn = pl.kernel(
        body,
        out_type=[jax.ShapeDtypeStruct((EPAD, C), jnp.float32)] * cbs,
        mesh=mesh,
        scratch_types=[
            pltpu.VMEM((K,), jnp.int32),
            pltpu.VMEM((K, C), jnp.float32),
            pltpu.SemaphoreType.DMA,
        ],
    )
    return fn(*hbs, srcr)


def _sc_scatter(m, dstr, zrow, din):
    cbs = din // C
    mesh = plsc.VectorSubcoreMesh(core_axis_name="c", subcore_axis_name="s")

    def body(*refs):
        m_ref = refs[:cbs]
        dstr_ref, zr_ref, out, didx, mbuf, acc, sem = refs[cbs:]
        cid = lax.axis_index("c")
        sid = lax.axis_index("s")
        wid = cid * 16 + sid

        row0_base = wid * EPT
        for cb in range(cbs):
            for j in range(RPT // K):
                pltpu.sync_copy(zr_ref, acc.at[pl.ds(sid * RPT + j * K, K)])
            plsc.subcore_barrier()

            def chunk(ch, carry):
                r0 = row0_base + ch * K
                pltpu.sync_copy(dstr_ref.at[wid, ch], didx)
                pltpu.sync_copy(m_ref[cb].at[pl.ds(r0, K)], mbuf)
                pltpu.sync_copy(mbuf, acc.at[didx], add=True)
                return carry
            lax.fori_loop(0, NCH, chunk, 0)
            plsc.subcore_barrier()
            pltpu.sync_copy(
                acc.at[pl.ds(sid * RPT, RPT)],
                out.at[cid, cb, pl.ds(sid * RPT, RPT)])

    fn = pl.kernel(
        body,
        out_type=jax.ShapeDtypeStruct((2, cbs, NACC, C), jnp.float32),
        mesh=mesh,
        scratch_types=[
            pltpu.VMEM((K,), jnp.int32),
            pltpu.VMEM((K, C), jnp.float32),
            pltpu.VMEM_SHARED((NACC, C), jnp.float32),
            pltpu.SemaphoreType.DMA,
        ],
    )
    return fn(*m, dstr, zrow)


def _msg_mlp(gbs, eap, We, be, din):
    cbs = din // C
    BE = 1024

    def body(*refs):
        grefs = refs[:cbs]
        ea_ref, We_ref, be_ref = refs[cbs:cbs + 3]
        m_ref = refs[cbs + 3:]
        g = jnp.concatenate([gr[...] for gr in grefs], axis=-1)
        e = jnp.dot(ea_ref[...], We_ref[...],
                    preferred_element_type=jnp.float32) + be_ref[...]
        mm = jnp.maximum(g + e, 0.0)
        for c2 in range(cbs):
            m_ref[c2][...] = mm[:, c2 * C:(c2 + 1) * C]

    return pl.pallas_call(
        body,
        grid=(EPAD // BE,),
        in_specs=[pl.BlockSpec((BE, C), lambda i: (i, 0))] * cbs + [
            pl.BlockSpec((BE, DE), lambda i: (i, 0)),
            pl.BlockSpec((DE, din), lambda i: (0, 0)),
            pl.BlockSpec((1, din), lambda i: (0, 0)),
        ],
        out_specs=[pl.BlockSpec((BE, C), lambda i: (i, 0))] * cbs,
        out_shape=[jax.ShapeDtypeStruct((EPAD, C), jnp.float32)] * cbs,
    )(*gbs, eap, We, be.reshape(1, -1))


def _mlp(hbs, parts, W1, b1, W2, b2, Wr, br, din, dout):
    cbs = din // C
    cbs2 = 2 * cbs

    def body(*refs):
        hrefs = refs[:cbs]
        p_ref, W1r, b1r, W2r, b2r, Wrr, brr = refs[cbs:cbs + 7]
        t_ref, r_ref, s1_ref = refs[cbs + 7:]
        i = pl.program_id(0)
        hv = jnp.concatenate([hr[...] for hr in hrefs], axis=-1)
        p = p_ref[...]
        aggr = jnp.concatenate(
            [p[0, c] + p[1, c] for c in range(cbs)], axis=-1)
        z = hv + aggr
        a = jnp.maximum(
            jnp.dot(z, W1r[...], preferred_element_type=jnp.float32)
            + b1r[...], 0.0)
        t = jnp.dot(a, W2r[...], preferred_element_type=jnp.float32) + b2r[...]
        r = jnp.dot(hv, Wrr[...], preferred_element_type=jnp.float32) + brr[...]
        t_ref[...] = t
        r_ref[...] = r
        p1 = jnp.sum(t.reshape(BN // 8, 8, dout), axis=0)

        @pl.when(i == 0)
        def _init():
            s1_ref[...] = jnp.zeros((8, dout), jnp.float32)
        s1_ref[...] += p1

    zero = lambda i: (0, 0)
    nb = lambda i: (i, 0)
    return pl.pallas_call(
        body,
        grid=(GN,),
        in_specs=[pl.BlockSpec((BN, C), nb)] * cbs + [
            pl.BlockSpec((2, cbs, BN, C), lambda i: (0, 0, i, 0)),
            pl.BlockSpec((din, dout), zero),
            pl.BlockSpec((1, dout), zero),
            pl.BlockSpec((dout, dout), zero),
            pl.BlockSpec((1, dout), zero),
            pl.BlockSpec((din, dout), zero),
            pl.BlockSpec((1, dout), zero),
        ],
        out_specs=[
            pl.BlockSpec((BN, dout), nb),
            pl.BlockSpec((BN, dout), nb),
            pl.BlockSpec((8, dout), zero),
        ],
        out_shape=[
            jax.ShapeDtypeStruct((N, dout), jnp.float32),
            jax.ShapeDtypeStruct((N, dout), jnp.float32),
            jax.ShapeDtypeStruct((8, dout), jnp.float32),
        ],
    )(*hbs, parts, W1, b1.reshape(1, -1), W2, b2.reshape(1, -1),
      Wr, br.reshape(1, -1))


def _bn_var(t, s1, dout):
    def body(t_ref, s1_ref, sv_ref):
        i = pl.program_id(0)
        mu = jnp.sum(s1_ref[...], axis=0) / N
        d = t_ref[...] - mu
        pv = jnp.sum((d * d).reshape(BN // 8, 8, dout), axis=0)

        @pl.when(i == 0)
        def _init():
            sv_ref[...] = jnp.zeros((8, dout), jnp.float32)
        sv_ref[...] += pv

    zero = lambda i: (0, 0)
    nb = lambda i: (i, 0)
    return pl.pallas_call(
        body,
        grid=(GN,),
        in_specs=[
            pl.BlockSpec((BN, dout), nb),
            pl.BlockSpec((8, dout), zero),
        ],
        out_specs=pl.BlockSpec((8, dout), zero),
        out_shape=jax.ShapeDtypeStruct((8, dout), jnp.float32),
    )(t, s1)


def _bn_residual(t, r, s1, s2, gamma, beta, dout, split):
    n_out = dout // C if split else 1
    out_w = C if split else dout

    def body(t_ref, r_ref, s1_ref, s2_ref, g_ref, b_ref, *outs):
        s1 = jnp.sum(s1_ref[...], axis=0)
        s2 = jnp.sum(s2_ref[...], axis=0)
        mu = s1 / N
        var = s2 / N
        inv = 1.0 / jnp.sqrt(var + 1e-5)
        y = jnp.maximum(
            g_ref[...] * (t_ref[...] - mu) * inv + b_ref[...] + r_ref[...],
            0.0)
        if split:
            for cb in range(n_out):
                outs[cb][...] = y[:, cb * C:(cb + 1) * C]
        else:
            outs[0][...] = y

    zero = lambda i: (0, 0)
    nb = lambda i: (i, 0)
    return pl.pallas_call(
        body,
        grid=(GN,),
        in_specs=[
            pl.BlockSpec((BN, dout), nb),
            pl.BlockSpec((BN, dout), nb),
            pl.BlockSpec((8, dout), zero),
            pl.BlockSpec((8, dout), zero),
            pl.BlockSpec((1, dout), zero),
            pl.BlockSpec((1, dout), zero),
        ],
        out_specs=[pl.BlockSpec((BN, out_w), nb)] * n_out,
        out_shape=[jax.ShapeDtypeStruct((N, out_w), jnp.float32)] * n_out,
    )(t, r, s1, s2, gamma.reshape(1, -1), beta.reshape(1, -1))


def _final(h, Wfc, bfc, g, b):
    H = Wfc.shape[1]

    def body(h_ref, W_ref, b_ref, g_ref, be_ref, o_ref):
        u = jnp.maximum(
            jnp.dot(h_ref[...], W_ref[...],
                    preferred_element_type=jnp.float32) + b_ref[...], 0.0)
        mu = jnp.mean(u, axis=1, keepdims=True)
        var = jnp.mean((u - mu) * (u - mu), axis=1, keepdims=True)
        o_ref[...] = g_ref[...] * (u - mu) / jnp.sqrt(var + 1e-5) + be_ref[...]

    zero = lambda i: (0, 0)
    return pl.pallas_call(
        body,
        grid=(GN,),
        in_specs=[
            pl.BlockSpec((BN, 64), lambda i: (i, 0)),
            pl.BlockSpec((64, H), zero),
            pl.BlockSpec((1, H), zero),
            pl.BlockSpec((1, H), zero),
            pl.BlockSpec((1, H), zero),
        ],
        out_specs=pl.BlockSpec((BN, H), lambda i: (i, 0)),
        out_shape=jax.ShapeDtypeStruct((N, H), jnp.float32),
    )(h, Wfc, bfc.reshape(1, -1), g.reshape(1, -1), b.reshape(1, -1))


DIMS_K = [(256, 1024), (1024, 512), (512, 128), (128, 64)]


def kernel(x, edge_index, edge_attr, params):
    src = edge_index[0].astype(jnp.int32)
    dst = edge_index[1].astype(jnp.int32)
    padi = (jnp.arange(NPADE, dtype=jnp.int32) % 240)
    srcp = jnp.concatenate([src, padi])
    dstp = jnp.concatenate([dst, padi + N])
    perm = jnp.argsort(dstp)
    srcs = srcp[perm].reshape(K, NT * NCH).T
    dsts = dstp[perm].reshape(K, NT * NCH).T
    srcr = srcs.reshape(NT, NCH, K)
    dstr = dsts.reshape(NT, NCH, K)
    eap = jnp.pad(edge_attr, ((0, NPADE), (0, 0)))[perm]
    eap = eap.reshape(K, NT * NCH, DE).transpose(1, 0, 2).reshape(EPAD, DE)
    zrow = jnp.zeros((K, C), jnp.float32)

    hbs = [x[:, i * C:(i + 1) * C] for i in range(256 // C)]
    for li, (din, dout) in enumerate(DIMS_K):
        p = params['conv%d' % (li + 1)]
        gbs = _sc_gather(hbs, srcr, din)
        m = _msg_mlp(gbs, eap, p['We'], p['be'], din)
        parts = _sc_scatter(m, dstr, zrow, din)
        t, r, s1 = _mlp(hbs, parts, p['W1'], p['b1'], p['W2'], p['b2'],
                        p['Wr'], p['br'], din, dout)
        sv = _bn_var(t, s1, dout)
        split = li < 3
        hbs = _bn_residual(t, r, s1, sv, p['gamma'], p['beta'], dout, split)

    return _final(hbs[0], params['Wfc'], params['bfc'],
                  params['ln_g'], params['ln_b'])


# speedup vs baseline: 1.3555x; 1.0627x over previous
import jax
import jax.numpy as jnp
from jax import lax
from jax.experimental import pallas as pl
from jax.experimental.pallas import tpu as pltpu
from jax.experimental.pallas import tpu_sc as plsc

N = 10000
E = 160000
DE = 16
C = 128
NT = 32
K = 128
NCH = 40
EPT = NCH * K
EPAD = NT * EPT
NPADE = EPAD - E
NACC = 10240
RPT = NACC // 16
BN = 400
GN = N // BN


def _sc_gather(hbs, srcr, din):
    cbs = din // C
    mesh = plsc.VectorSubcoreMesh(core_axis_name="c", subcore_axis_name="s")

    def body(*refs):
        hb = refs[:cbs]
        srcr_ref = refs[cbs]
        outs = refs[cbs + 1:2 * cbs + 1]
        sidx, gbuf, sem = refs[2 * cbs + 1:]

        cid = lax.axis_index("c")
        sid = lax.axis_index("s")
        wid = cid * 16 + sid
        row0_base = wid * EPT
        for cb in range(cbs):
            def chunk(ch, carry):
                pltpu.sync_copy(srcr_ref.at[wid, ch], sidx)
                pltpu.async_copy(hb[cb].at[sidx], gbuf, sem).wait()
                pltpu.sync_copy(
                    gbuf, outs[cb].at[pl.ds(row0_base + ch * K, K)])
                return carry
            lax.fori_loop(0, NCH, chunk, 0)

    fn = pl.kernel(
        body,
        out_type=[jax.ShapeDtypeStruct((EPAD, C), jnp.float32)] * cbs,
        mesh=mesh,
        scratch_types=[
            pltpu.VMEM((K,), jnp.int32),
            pltpu.VMEM((K, C), jnp.float32),
            pltpu.SemaphoreType.DMA,
        ],
    )
    return fn(*hbs, srcr)


def _sc_scatter(m, dstr, zrow, din):
    cbs = din // C
    mesh = plsc.VectorSubcoreMesh(core_axis_name="c", subcore_axis_name="s")

    def body(*refs):
        m_ref = refs[:cbs]
        dstr_ref, zr_ref, out, didx, mbuf, acc, sem = refs[cbs:]
        cid = lax.axis_index("c")
        sid = lax.axis_index("s")
        wid = cid * 16 + sid

        row0_base = wid * EPT
        for cb in range(cbs):
            for j in range(RPT // K):
                pltpu.sync_copy(zr_ref, acc.at[pl.ds(sid * RPT + j * K, K)])
            plsc.subcore_barrier()

            def chunk(ch, carry):
                r0 = row0_base + ch * K
                pltpu.sync_copy(dstr_ref.at[wid, ch], didx)
                pltpu.sync_copy(m_ref[cb].at[pl.ds(r0, K)], mbuf)
                pltpu.sync_copy(mbuf, acc.at[didx], add=True)
                return carry
            lax.fori_loop(0, NCH, chunk, 0)
            plsc.subcore_barrier()
            pltpu.sync_copy(
                acc.at[pl.ds(sid * RPT, RPT)],
                out.at[cid, cb, pl.ds(sid * RPT, RPT)])

    fn = pl.kernel(
        body,
        out_type=jax.ShapeDtypeStruct((2, cbs, NACC, C), jnp.float32),
        mesh=mesh,
        scratch_types=[
            pltpu.VMEM((K,), jnp.int32),
            pltpu.VMEM((K, C), jnp.float32),
            pltpu.VMEM_SHARED((NACC, C), jnp.float32),
            pltpu.SemaphoreType.DMA,
        ],
    )
    return fn(*m, dstr, zrow)


def _msg_mlp(gbs, eap, We, be, din):
    cbs = din // C
    BE = 1024

    def body(*refs):
        grefs = refs[:cbs]
        ea_ref, We_ref, be_ref = refs[cbs:cbs + 3]
        m_ref = refs[cbs + 3:]
        g = jnp.concatenate([gr[...] for gr in grefs], axis=-1)
        e = jnp.dot(ea_ref[...], We_ref[...],
                    preferred_element_type=jnp.float32) + be_ref[...]
        mm = jnp.maximum(g + e, 0.0)
        for c2 in range(cbs):
            m_ref[c2][...] = mm[:, c2 * C:(c2 + 1) * C]

    return pl.pallas_call(
        body,
        grid=(EPAD // BE,),
        in_specs=[pl.BlockSpec((BE, C), lambda i: (i, 0))] * cbs + [
            pl.BlockSpec((BE, DE), lambda i: (i, 0)),
            pl.BlockSpec((DE, din), lambda i: (0, 0)),
            pl.BlockSpec((1, din), lambda i: (0, 0)),
        ],
        out_specs=[pl.BlockSpec((BE, C), lambda i: (i, 0))] * cbs,
        out_shape=[jax.ShapeDtypeStruct((EPAD, C), jnp.float32)] * cbs,
    )(*gbs, eap, We, be.reshape(1, -1))


def _mlp(hbs, parts, W1, b1, W2, b2, Wr, br, din, dout):
    cbs = din // C
    cbs2 = 2 * cbs

    def body(*refs):
        hrefs = refs[:cbs]
        p_ref, W1r, b1r, W2r, b2r, Wrr, brr = refs[cbs:cbs + 7]
        t_ref, r_ref, s1_ref = refs[cbs + 7:]
        i = pl.program_id(0)
        hv = jnp.concatenate([hr[...] for hr in hrefs], axis=-1)
        p = p_ref[...]
        aggr = jnp.concatenate(
            [p[0, c] + p[1, c] for c in range(cbs)], axis=-1)
        z = hv + aggr
        a = jnp.maximum(
            jnp.dot(z, W1r[...], preferred_element_type=jnp.float32)
            + b1r[...], 0.0)
        t = jnp.dot(a, W2r[...], preferred_element_type=jnp.float32) + b2r[...]
        r = jnp.dot(hv, Wrr[...], preferred_element_type=jnp.float32) + brr[...]
        t_ref[...] = t
        r_ref[...] = r
        p1 = jnp.sum(t.reshape(BN // 8, 8, dout), axis=0)

        @pl.when(i == 0)
        def _init():
            s1_ref[...] = jnp.zeros((8, dout), jnp.float32)
        s1_ref[...] += p1

    zero = lambda i: (0, 0)
    nb = lambda i: (i, 0)
    return pl.pallas_call(
        body,
        grid=(GN,),
        in_specs=[pl.BlockSpec((BN, C), nb)] * cbs + [
            pl.BlockSpec((2, cbs, BN, C), lambda i: (0, 0, i, 0)),
            pl.BlockSpec((din, dout), zero),
            pl.BlockSpec((1, dout), zero),
            pl.BlockSpec((dout, dout), zero),
            pl.BlockSpec((1, dout), zero),
            pl.BlockSpec((din, dout), zero),
            pl.BlockSpec((1, dout), zero),
        ],
        out_specs=[
            pl.BlockSpec((BN, dout), nb),
            pl.BlockSpec((BN, dout), nb),
            pl.BlockSpec((8, dout), zero),
        ],
        out_shape=[
            jax.ShapeDtypeStruct((N, dout), jnp.float32),
            jax.ShapeDtypeStruct((N, dout), jnp.float32),
            jax.ShapeDtypeStruct((8, dout), jnp.float32),
        ],
    )(*hbs, parts, W1, b1.reshape(1, -1), W2, b2.reshape(1, -1),
      Wr, br.reshape(1, -1))


def _bn_var(t, s1, dout):
    def body(t_ref, s1_ref, sv_ref):
        i = pl.program_id(0)
        mu = jnp.sum(s1_ref[...], axis=0) / N
        d = t_ref[...] - mu
        pv = jnp.sum((d * d).reshape(BN // 8, 8, dout), axis=0)

        @pl.when(i == 0)
        def _init():
            sv_ref[...] = jnp.zeros((8, dout), jnp.float32)
        sv_ref[...] += pv

    zero = lambda i: (0, 0)
    nb = lambda i: (i, 0)
    return pl.pallas_call(
        body,
        grid=(GN,),
        in_specs=[
            pl.BlockSpec((BN, dout), nb),
            pl.BlockSpec((8, dout), zero),
        ],
        out_specs=pl.BlockSpec((8, dout), zero),
        out_shape=jax.ShapeDtypeStruct((8, dout), jnp.float32),
    )(t, s1)


def _bn_residual(t, r, s1, s2, gamma, beta, dout, split):
    n_out = dout // C if split else 1
    out_w = C if split else dout

    def body(t_ref, r_ref, s1_ref, s2_ref, g_ref, b_ref, *outs):
        s1 = jnp.sum(s1_ref[...], axis=0)
        s2 = jnp.sum(s2_ref[...], axis=0)
        mu = s1 / N
        var = s2 / N
        inv = 1.0 / jnp.sqrt(var + 1e-5)
        y = jnp.maximum(
            g_ref[...] * (t_ref[...] - mu) * inv + b_ref[...] + r_ref[...],
            0.0)
        if split:
            for cb in range(n_out):
                outs[cb][...] = y[:, cb * C:(cb + 1) * C]
        else:
            outs[0][...] = y

    zero = lambda i: (0, 0)
    nb = lambda i: (i, 0)
    return pl.pallas_call(
        body,
        grid=(GN,),
        in_specs=[
            pl.BlockSpec((BN, dout), nb),
            pl.BlockSpec((BN, dout), nb),
            pl.BlockSpec((8, dout), zero),
            pl.BlockSpec((8, dout), zero),
            pl.BlockSpec((1, dout), zero),
            pl.BlockSpec((1, dout), zero),
        ],
        out_specs=[pl.BlockSpec((BN, out_w), nb)] * n_out,
        out_shape=[jax.ShapeDtypeStruct((N, out_w), jnp.float32)] * n_out,
    )(t, r, s1, s2, gamma.reshape(1, -1), beta.reshape(1, -1))


def _final(h, Wfc, bfc, g, b):
    H = Wfc.shape[1]

    def body(h_ref, W_ref, b_ref, g_ref, be_ref, o_ref):
        u = jnp.maximum(
            jnp.dot(h_ref[...], W_ref[...],
                    preferred_element_type=jnp.float32) + b_ref[...], 0.0)
        mu = jnp.mean(u, axis=1, keepdims=True)
        var = jnp.mean((u - mu) * (u - mu), axis=1, keepdims=True)
        o_ref[...] = g_ref[...] * (u - mu) / jnp.sqrt(var + 1e-5) + be_ref[...]

    zero = lambda i: (0, 0)
    return pl.pallas_call(
        body,
        grid=(GN,),
        in_specs=[
            pl.BlockSpec((BN, 64), lambda i: (i, 0)),
            pl.BlockSpec((64, H), zero),
            pl.BlockSpec((1, H), zero),
            pl.BlockSpec((1, H), zero),
            pl.BlockSpec((1, H), zero),
        ],
        out_specs=pl.BlockSpec((BN, H), lambda i: (i, 0)),
        out_shape=jax.ShapeDtypeStruct((N, H), jnp.float32),
    )(h, Wfc, bfc.reshape(1, -1), g.reshape(1, -1), b.reshape(1, -1))


DIMS_K = [(256, 1024), (1024, 512), (512, 128), (128, 64)]


def kernel(x, edge_index, edge_attr, params):
    src = edge_index[0].astype(jnp.int32)
    dst = edge_index[1].astype(jnp.int32)
    padi = (jnp.arange(NPADE, dtype=jnp.int32) % 240)
    srcr = jnp.concatenate([src, padi]).reshape(NT, NCH, K)
    dstr = jnp.concatenate([dst, padi + N]).reshape(NT, NCH, K)
    eap = jnp.pad(edge_attr, ((0, NPADE), (0, 0)))
    zrow = jnp.zeros((K, C), jnp.float32)

    hbs = [x[:, i * C:(i + 1) * C] for i in range(256 // C)]
    for li, (din, dout) in enumerate(DIMS_K):
        p = params['conv%d' % (li + 1)]
        gbs = _sc_gather(hbs, srcr, din)
        m = _msg_mlp(gbs, eap, p['We'], p['be'], din)
        parts = _sc_scatter(m, dstr, zrow, din)
        t, r, s1 = _mlp(hbs, parts, p['W1'], p['b1'], p['W2'], p['b2'],
                        p['Wr'], p['br'], din, dout)
        sv = _bn_var(t, s1, dout)
        split = li < 3
        hbs = _bn_residual(t, r, s1, sv, p['gamma'], p['beta'], dout, split)

    return _final(hbs[0], params['Wfc'], params['bfc'],
                  params['ln_g'], params['ln_b'])
